# baseline (speedup 1.0000x reference)
import jax
import jax.numpy as jnp
from jax import lax
from jax.experimental import pallas as pl
from jax.experimental.pallas import tpu as pltpu
from jax.experimental.pallas import tpu_sc as plsc

N = 48000
E = 576000
NC = 2
NS = 16
NW = NC * NS
K = 128
EPW = 18432
EPAD = NW * EPW - E
SL = 6144
NCH = 12
CH = 4000
CAP = 1792
NBB = CAP // K
STRIPE = CH // NS
ZR = 125

TN = 480

_BN_S = float(1.0 / (1.0 + 1e-5) ** 0.5)

_MESH = dict(core_axis_name="c", subcore_axis_name="s",
             num_cores=NC, num_subcores=NS)
_SC_PARAMS = None


def _sc_kwargs():
  return dict(
      mesh=plsc.VectorSubcoreMesh(**_MESH),
      compiler_params=pltpu.CompilerParams(use_tc_tiling_on_sc=False,
                                           needs_layout_passes=False),
  )


def _partition_sc():
  def body(src_h, dst_h, zer_h, tra_h, sb_h, db_h, sf_h, df_h, cnt_h,
           *bufs):
    sstrip, dstrip = bufs[0], bufs[1]
    sbkts = bufs[2:2 + NCH]
    dbkts = bufs[2 + NCH:2 + 2 * NCH]
    cnt = bufs[2 + 2 * NCH]
    cid = lax.axis_index("c")
    tid = lax.axis_index("s")
    wid = cid * NS + tid

    for c in range(NCH):
      pltpu.sync_copy(zer_h, sbkts[c])
      pltpu.sync_copy(tra_h, dbkts[c])

    def scan(i, offs):
      sv = sstrip[pl.ds(i * 16, 16)]
      dv = dstrip[pl.ds(i * 16, 16)]
      new = []
      for c in range(NCH):
        m = (dv >= c * CH) & (dv < (c + 1) * CH)
        rel = dv - c * CH
        mi = m.astype(jnp.int32)
        cs = plsc.cumsum(mi)
        pos = jnp.minimum(offs[c] + cs - mi, CAP - 1)
        plsc.store_scatter(sbkts[c], [pos], sv, mask=m)
        plsc.store_scatter(dbkts[c], [pos], rel, mask=m)
        new.append(offs[c] + plsc.all_reduce_population_count(m))
      return tuple(new)

    offs = tuple(jnp.zeros((16,), jnp.int32) for _ in range(NCH))
    for s in range(EPW // SL):
      pltpu.sync_copy(src_h.at[wid].at[pl.ds(s * SL, SL)], sstrip)
      pltpu.sync_copy(dst_h.at[wid].at[pl.ds(s * SL, SL)], dstrip)
      offs = lax.fori_loop(0, SL // 16, scan, offs)

    for c in range(NCH):
      for j in range(NBB):
        pltpu.sync_copy(sbkts[c].at[pl.ds(j * K, K)],
                        sb_h.at[wid].at[c].at[j])
        pltpu.sync_copy(dbkts[c].at[pl.ds(j * K, K)],
                        db_h.at[wid].at[c].at[j])
      pltpu.sync_copy(sbkts[c], sf_h.at[c].at[pl.ds(wid * CAP, CAP)])
      pltpu.sync_copy(dbkts[c], df_h.at[c].at[pl.ds(wid * CAP, CAP)])
    for c in range(NCH):
      cnt[c, pl.ds(0, 16)] = offs[c]
    pltpu.sync_copy(cnt, cnt_h.at[wid])

  return pl.kernel(
      body,
      out_type=[jax.ShapeDtypeStruct((NW, NCH, NBB, K), jnp.int32),
                jax.ShapeDtypeStruct((NW, NCH, NBB, K), jnp.int32),
                jax.ShapeDtypeStruct((NCH, NW * CAP), jnp.int32),
                jax.ShapeDtypeStruct((NCH, NW * CAP), jnp.int32),
                jax.ShapeDtypeStruct((NW, NCH, 16), jnp.int32)],
      scratch_types=(
          [pltpu.VMEM((SL,), jnp.int32)] * 2
          + [pltpu.VMEM((CAP,), jnp.int32)] * (2 * NCH)
          + [pltpu.VMEM((NCH, 16), jnp.int32)]
      ),
      **_sc_kwargs(),
  )


def _seg_sc(w):
  def body(g_h, sb_h, db_h, cnt_h, z_h, out, sb, db, cntv, rows0, rows1,
           zbuf, acc, gs0, gs1):
    cid = lax.axis_index("c")
    tid = lax.axis_index("s")
    wid = cid * NS + tid

    pltpu.sync_copy(sb_h.at[wid], sb)
    pltpu.sync_copy(db_h.at[wid], db)
    pltpu.sync_copy(cnt_h.at[wid], cntv)
    pltpu.sync_copy(z_h, zbuf)

    for c in range(NCH):
      for z in range(STRIPE // ZR):
        pltpu.sync_copy(zbuf, acc.at[pl.ds(tid * STRIPE + z * ZR, ZR)])

      sbc = sb.at[c]
      dbc = db.at[c]
      plsc.subcore_barrier()

      def batch(j, _):
        pltpu.async_copy(g_h.at[sbc.at[j]], rows0, gs0)
        pltpu.make_async_copy(g_h.at[sbc.at[j]], rows0, gs0).wait()
        pltpu.sync_copy(rows0, acc.at[dbc.at[j]], add=True)
        return _

      cvec = cntv[c, pl.ds(0, 16)]
      nb = jnp.minimum((jnp.max(cvec) + (K - 1)) // K, NBB)
      lax.fori_loop(0, nb, batch, None)
      plsc.subcore_barrier()
      pltpu.sync_copy(acc.at[pl.ds(tid * STRIPE, STRIPE)],
                      out.at[cid].at[pl.ds(c * CH + tid * STRIPE, STRIPE)])

  return pl.kernel(
      body,
      out_type=jax.ShapeDtypeStruct((NC, N, w), jnp.float32),
      scratch_types=[
          pltpu.VMEM((NCH, NBB, K), jnp.int32),
          pltpu.VMEM((NCH, NBB, K), jnp.int32),
          pltpu.VMEM((NCH, 16), jnp.int32),
          pltpu.VMEM((K, w), jnp.float32),
          pltpu.VMEM((K, w), jnp.float32),
          pltpu.VMEM((ZR, w), jnp.float32),
          pltpu.VMEM_SHARED((CH + 8, w), jnp.float32),
          pltpu.SemaphoreType.DMA,
          pltpu.SemaphoreType.DMA,
      ],
      **_sc_kwargs(),
  )


def _seg_tile(G):
  R = NW // G
  NWR = G
  NBT = NWR * NBB
  SBN = NWR * CAP

  def body(gt_h, sf_h, df_h, out, sb1, db1, rb0, rb1, acc, gs0, gs1):
    cid = lax.axis_index("c")
    tid = lax.axis_index("s")
    wid = cid * NS + tid
    g_ = wid // R
    r_ = wid % R
    rb = (rb0, rb1)
    gs = (gs0, gs1)

    for c in range(NCH):
      pltpu.sync_copy(sf_h.at[c].at[pl.ds(r_ * SBN, SBN)], sb1)
      pltpu.sync_copy(df_h.at[c].at[pl.ds(r_ * SBN, SBN)], db1)

      def adj(i, _):
        v = sb1[pl.ds(i * 16, 16)]
        sb1[pl.ds(i * 16, 16)] = v + jnp.broadcast_to(g_ * N, (16,))
        return _

      lax.fori_loop(0, SBN // 16, adj, None)

      def zero(i, _):
        acc[i, pl.ds(0, 16)] = jnp.zeros((16,), jnp.float32)
        return _

      lax.fori_loop(0, CH + 8, zero, None)

      pltpu.async_copy(gt_h.at[sb1.at[pl.ds(0, K)]], rb0, gs0)
      pltpu.async_copy(gt_h.at[sb1.at[pl.ds(K, K)]], rb1, gs1)

      def batch(b2, _):
        for b in range(2):
          bb = b2 * 2 + b
          pltpu.make_async_copy(gt_h.at[sb1.at[pl.ds(0, K)]], rb[b],
                                gs[b]).wait()
          for e16 in range(8):
            dvec = db1[pl.ds(bb * K + e16 * 16, 16)]
            rvec = jnp.full((16,), e16 * 16, jnp.int32) + lax.iota(
                jnp.int32, 16)
            for col in range(16):
              cvec = jnp.full((16,), col, jnp.int32)
              x = plsc.load_gather(rb[b], [rvec, cvec])
              plsc.addupdate_scatter(acc, [dvec, cvec], x)
          bn = jnp.minimum(bb + 2, NBT - 1)
          pltpu.async_copy(gt_h.at[sb1.at[pl.ds(bn * K, K)]], rb[b], gs[b])
        return _

      lax.fori_loop(0, NBT // 2, batch, None)
      for b in range(2):
        pltpu.make_async_copy(gt_h.at[sb1.at[pl.ds(0, K)]], rb[b],
                              gs[b]).wait()
      pltpu.sync_copy(acc.at[pl.ds(0, CH)],
                      out.at[r_].at[g_].at[pl.ds(c * CH, CH)])

  return pl.kernel(
      body,
      out_type=jax.ShapeDtypeStruct((R, G, N, 16), jnp.float32),
      scratch_types=[
          pltpu.VMEM((SBN,), jnp.int32),
          pltpu.VMEM((SBN,), jnp.int32),
          pltpu.VMEM((K, 16), jnp.float32),
          pltpu.VMEM((K, 16), jnp.float32),
          pltpu.VMEM((CH + 8, 16), jnp.float32),
          pltpu.SemaphoreType.DMA,
          pltpu.SemaphoreType.DMA,
      ],
      **_sc_kwargs(),
  )


def _deg_sc():
  w = 8

  def body(ones_h, db_h, cnt_h, z_h, out, db, cntv, rows0, zbuf, acc):
    cid = lax.axis_index("c")
    tid = lax.axis_index("s")
    wid = cid * NS + tid

    pltpu.sync_copy(db_h.at[wid], db)
    pltpu.sync_copy(cnt_h.at[wid], cntv)
    pltpu.sync_copy(ones_h, rows0)
    pltpu.sync_copy(z_h, zbuf)

    for c in range(NCH):
      for z in range(STRIPE // ZR):
        pltpu.sync_copy(zbuf, acc.at[pl.ds(tid * STRIPE + z * ZR, ZR)])

      dbc = db.at[c]
      plsc.subcore_barrier()

      def batch(j, _):
        pltpu.sync_copy(rows0, acc.at[dbc.at[j]], add=True)
        return _

      cvec = cntv[c, pl.ds(0, 16)]
      nb = jnp.minimum((jnp.max(cvec) + (K - 1)) // K, NBB)
      lax.fori_loop(0, nb, batch, None)
      plsc.subcore_barrier()
      pltpu.sync_copy(acc.at[pl.ds(tid * STRIPE, STRIPE)],
                      out.at[cid].at[pl.ds(c * CH + tid * STRIPE, STRIPE)])

  return pl.kernel(
      body,
      out_type=jax.ShapeDtypeStruct((NC, N, w), jnp.float32),
      scratch_types=[
          pltpu.VMEM((NCH, NBB, K), jnp.int32),
          pltpu.VMEM((NCH, 16), jnp.int32),
          pltpu.VMEM((K, w), jnp.float32),
          pltpu.VMEM((ZR, w), jnp.float32),
          pltpu.VMEM_SHARED((CH + 8, w), jnp.float32),
      ],
      **_sc_kwargs(),
  )


def _dot(a, b):
  return lax.dot_general(a, b, (((1,), (0,)), ((), ())),
                         preferred_element_type=jnp.float32)


def _ffn_a(x_ref, w_ref, a_ref, c_ref, o_ref):
  y = _dot(x_ref[...], w_ref[...]) * a_ref[...] + c_ref[...]
  o_ref[...] = jnp.maximum(y, 0.0)


def _ffn_b(h_ref, w_ref, a_ref, c_ref, o_ref):
  o_ref[...] = _dot(h_ref[...], w_ref[...]) * a_ref[...] + c_ref[...]


def _dinv_body(d_ref, o_ref):
  o_ref[...] = lax.rsqrt(d_ref[0] + d_ref[1] + 1.0)


def _mk_pre(G_out):
  def body(h_ref, dv_ref, wc_ref, o_ref):
    dv = dv_ref[:, 0:1]
    for g in range(G_out):
      o_ref[g] = _dot(h_ref[...], wc_ref[g]) * dv
  return body


def _mk_combine(R_in, G_in):
  def body(s_ref, g_ref, dv_ref, bc_ref, wc_ref, o_ref):
    dv = dv_ref[:, 0:1]
    cols = []
    for g in range(G_in):
      sg = s_ref[0, g]
      for r in range(1, R_in):
        sg = sg + s_ref[r, g]
      cols.append(sg + g_ref[g])
    sg_all = jnp.concatenate(cols, axis=1) if G_in > 1 else cols[0]
    a = jnp.maximum(sg_all * dv + bc_ref[...], 0.0)
    G_out = o_ref.shape[0]
    for go in range(G_out):
      o_ref[go] = _dot(a, wc_ref[go]) * dv

  return body


def _combine_body(s_ref, g_ref, dv_ref, bc_ref, wc_ref, o_ref):
  dv = dv_ref[:, 0:1]
  a = jnp.maximum((s_ref[0] + s_ref[1] + g_ref[...]) * dv + bc_ref[...], 0.0)
  o_ref[...] = _dot(a, wc_ref[...]) * dv


def _last_body(s_ref, g_ref, dv_ref, bc_ref, o_ref):
  dv = dv_ref[:, 0:1]
  o_ref[...] = jnp.maximum(
      (s_ref[0] + s_ref[1] + g_ref[...]) * dv + bc_ref[...], 0.0)


def _final_mm(a_ref, w_ref, b_ref, o_ref):
  o_ref[...] = _dot(a_ref[...], w_ref[...]) + b_ref[...]


def _pad2(m, rows, cols):
  r, c = m.shape
  if r == rows and c == cols:
    return m
  return jnp.pad(m, ((0, rows - r), (0, cols - c)))


def _full_spec(shape):
  return pl.BlockSpec(shape, lambda i: tuple(0 for _ in shape))


def _row_spec(width):
  return pl.BlockSpec((TN, width), lambda i: (i, 0))


def _s_spec(width):
  return pl.BlockSpec((NC, TN, width), lambda i: (0, i, 0))


def kernel(x, edge_index, W1, b1, g1, be1, W2, b2, g2, be2,
           Wc1, bc1, Wc2, bc2, Wc3, bc3, Wc4, bc4, Wc5, bc5, fcW, fcb):
  pad_src = jnp.zeros((EPAD,), jnp.int32)
  pad_dst = jnp.full((EPAD,), -1, jnp.int32)
  src2 = jnp.concatenate([edge_index[0], pad_src]).reshape(NW, EPW)
  dst2 = jnp.concatenate([edge_index[1], pad_dst]).reshape(NW, EPW)
  zer_i = jnp.zeros((CAP,), jnp.int32)
  tra_i = CH + (jnp.arange(CAP, dtype=jnp.int32) % 8)

  sb_h, db_h, sf_h, df_h, cnt_h = _partition_sc()(src2, dst2, zer_i, tra_i)

  def seg(w, g):
    z = jnp.zeros((ZR, w), jnp.float32)
    return _seg_sc(w)(g, sb_h, db_h, cnt_h, z)

  a1 = (g1 * _BN_S).reshape(1, 1000)
  c1 = (b1 * g1 * _BN_S + be1).reshape(1, 1000)
  a2 = _pad2((g2 * _BN_S).reshape(1, 250), 1, 256)
  c2 = _pad2((b2 * g2 * _BN_S + be2).reshape(1, 250), 1, 256)

  xp = _pad2(x, N, 256)
  W1p = _pad2(W1, 256, 1000)
  W2p = _pad2(W2, 1000, 256)
  Wc1p = _pad2(Wc1, 256, 128)

  degp = _deg_sc()(jnp.ones((K, 8), jnp.float32), db_h, cnt_h,
                   jnp.zeros((ZR, 8), jnp.float32))
  dinv = pl.pallas_call(
      _dinv_body,
      grid=(N // TN,),
      in_specs=[_s_spec(8)],
      out_specs=pl.BlockSpec((TN, 8), lambda i: (i, 0)),
      out_shape=jax.ShapeDtypeStruct((N, 8), jnp.float32),
  )(degp)

  h = pl.pallas_call(
      _ffn_a,
      grid=(N // TN,),
      in_specs=[_row_spec(256), _full_spec((256, 1000)),
                _full_spec((1, 1000)), _full_spec((1, 1000))],
      out_specs=pl.BlockSpec((TN, 1000), lambda i: (i, 0)),
      out_shape=jax.ShapeDtypeStruct((N, 1000), jnp.float32),
  )(xp, W1p, a1, c1)

  h2 = pl.pallas_call(
      _ffn_b,
      grid=(N // TN,),
      in_specs=[_row_spec(1000), _full_spec((1000, 256)),
                _full_spec((1, 256)), _full_spec((1, 256))],
      out_specs=pl.BlockSpec((TN, 256), lambda i: (i, 0)),
      out_shape=jax.ShapeDtypeStruct((N, 256), jnp.float32),
  )(h, W2p, a2, c2)

  g = pl.pallas_call(
      _mk_pre(8),
      grid=(N // TN,),
      in_specs=[_row_spec(256), _row_spec(8), _full_spec((8, 256, 16))],
      out_specs=pl.BlockSpec((8, TN, 16), lambda i: (0, i, 0)),
      out_shape=jax.ShapeDtypeStruct((8, N, 16), jnp.float32),
  )(h2, dinv, Wc1p.reshape(256, 8, 16).transpose(1, 0, 2))
  g = g.transpose(1, 0, 2).reshape(N, 128)

  layer_cfg = [
      (128, bc1, Wc2, 64),
      (64, bc2, Wc3, 32),
      (32, bc3, Wc4, 16),
      (16, bc4, Wc5, 8),
  ]
  for d_in, bc, wc, d_out in layer_cfg:
    sp = seg(d_in, g)
    g = pl.pallas_call(
        _combine_body,
        grid=(N // TN,),
        in_specs=[_s_spec(d_in), _row_spec(d_in), _row_spec(8),
                  _full_spec((1, d_in)), _full_spec((d_in, d_out))],
        out_specs=_row_spec(d_out),
        out_shape=jax.ShapeDtypeStruct((N, d_out), jnp.float32),
    )(sp, g, dinv, bc.reshape(1, d_in), wc)
  g5 = g

  s5 = seg(8, g5)
  u5 = pl.pallas_call(
      _last_body,
      grid=(N // TN,),
      in_specs=[_s_spec(8), _row_spec(8), _row_spec(8), _full_spec((1, 8))],
      out_specs=_row_spec(8),
      out_shape=jax.ShapeDtypeStruct((N, 8), jnp.float32),
  )(s5, g5, dinv, bc5.reshape(1, 8))

  ar = u5.reshape(320, 1200)
  fcWp = _pad2(fcW, 1200, 128)
  fcbp = _pad2(fcb.reshape(1, 4), 1, 128)
  out = pl.pallas_call(
      _final_mm,
      grid=(1,),
      in_specs=[_full_spec((320, 1200)), _full_spec((1200, 128)),
                _full_spec((1, 128))],
      out_specs=pl.BlockSpec((320, 128), lambda i: (0, 0)),
      out_shape=jax.ShapeDtypeStruct((320, 128), jnp.float32),
  )(ar, fcWp, fcbp)
  return out[:, :4]


# speedup vs baseline: 1.0995x; 1.0995x over previous
import jax
import jax.numpy as jnp
from jax import lax
from jax.experimental import pallas as pl
from jax.experimental.pallas import tpu as pltpu
from jax.experimental.pallas import tpu_sc as plsc

N = 48000
E = 576000
NC = 2
NS = 16
NW = NC * NS
K = 128
EPW = 18432
EPAD = NW * EPW - E
SL = 6144
NCH = 12
CH = 4000
CAP = 1792
NBB = CAP // K
STRIPE = CH // NS
ZR = 125

TN = 480

_BN_S = float(1.0 / (1.0 + 1e-5) ** 0.5)

_MESH = dict(core_axis_name="c", subcore_axis_name="s",
             num_cores=NC, num_subcores=NS)
_SC_PARAMS = None


def _sc_kwargs():
  return dict(
      mesh=plsc.VectorSubcoreMesh(**_MESH),
      compiler_params=pltpu.CompilerParams(use_tc_tiling_on_sc=False,
                                           needs_layout_passes=False),
  )


def _partition_sc():
  def body(src_h, dst_h, zer_h, tra_h, sb_h, db_h, sf_h, df_h, cnt_h,
           *bufs):
    sstrip, dstrip = bufs[0], bufs[1]
    sbkts = bufs[2:2 + NCH]
    dbkts = bufs[2 + NCH:2 + 2 * NCH]
    cnt = bufs[2 + 2 * NCH]
    cid = lax.axis_index("c")
    tid = lax.axis_index("s")
    wid = cid * NS + tid

    for c in range(NCH):
      pltpu.sync_copy(zer_h, sbkts[c])
      pltpu.sync_copy(tra_h, dbkts[c])

    def scan(i, offs):
      sv = sstrip[pl.ds(i * 16, 16)]
      dv = dstrip[pl.ds(i * 16, 16)]
      new = []
      for c in range(NCH):
        m = (dv >= c * CH) & (dv < (c + 1) * CH)
        rel = dv - c * CH
        mi = m.astype(jnp.int32)
        cs = plsc.cumsum(mi)
        pos = jnp.minimum(offs[c] + cs - mi, CAP - 1)
        plsc.store_scatter(sbkts[c], [pos], sv, mask=m)
        plsc.store_scatter(dbkts[c], [pos], rel, mask=m)
        new.append(offs[c] + plsc.all_reduce_population_count(m))
      return tuple(new)

    offs = tuple(jnp.zeros((16,), jnp.int32) for _ in range(NCH))
    for s in range(EPW // SL):
      pltpu.sync_copy(src_h.at[wid].at[pl.ds(s * SL, SL)], sstrip)
      pltpu.sync_copy(dst_h.at[wid].at[pl.ds(s * SL, SL)], dstrip)
      offs = lax.fori_loop(0, SL // 16, scan, offs)

    for c in range(NCH):
      for j in range(NBB):
        pltpu.sync_copy(sbkts[c].at[pl.ds(j * K, K)],
                        sb_h.at[wid].at[c].at[j])
        pltpu.sync_copy(dbkts[c].at[pl.ds(j * K, K)],
                        db_h.at[wid].at[c].at[j])
      pltpu.sync_copy(sbkts[c], sf_h.at[c].at[pl.ds(wid * CAP, CAP)])
      pltpu.sync_copy(dbkts[c], df_h.at[c].at[pl.ds(wid * CAP, CAP)])
    for c in range(NCH):
      cnt[c, pl.ds(0, 16)] = offs[c]
    pltpu.sync_copy(cnt, cnt_h.at[wid])

  return pl.kernel(
      body,
      out_type=[jax.ShapeDtypeStruct((NW, NCH, NBB, K), jnp.int32),
                jax.ShapeDtypeStruct((NW, NCH, NBB, K), jnp.int32),
                jax.ShapeDtypeStruct((NCH, NW * CAP), jnp.int32),
                jax.ShapeDtypeStruct((NCH, NW * CAP), jnp.int32),
                jax.ShapeDtypeStruct((NW, NCH, 16), jnp.int32)],
      scratch_types=(
          [pltpu.VMEM((SL,), jnp.int32)] * 2
          + [pltpu.VMEM((CAP,), jnp.int32)] * (2 * NCH)
          + [pltpu.VMEM((NCH, 16), jnp.int32)]
      ),
      **_sc_kwargs(),
  )


def _seg_sc(w):
  def body(g_h, sb_h, db_h, cnt_h, z_h, out, sb, db, cntv, rows0, rows1,
           zbuf, acc, gs0, gs1):
    cid = lax.axis_index("c")
    tid = lax.axis_index("s")
    wid = cid * NS + tid

    pltpu.sync_copy(sb_h.at[wid], sb)
    pltpu.sync_copy(db_h.at[wid], db)
    pltpu.sync_copy(cnt_h.at[wid], cntv)
    pltpu.sync_copy(z_h, zbuf)

    for c in range(NCH):
      for z in range(STRIPE // ZR):
        pltpu.sync_copy(zbuf, acc.at[pl.ds(tid * STRIPE + z * ZR, ZR)])

      sbc = sb.at[c]
      dbc = db.at[c]
      plsc.subcore_barrier()

      def batch(j, _):
        pltpu.async_copy(g_h.at[sbc.at[j]], rows0, gs0)
        pltpu.make_async_copy(g_h.at[sbc.at[j]], rows0, gs0).wait()
        pltpu.sync_copy(rows0, acc.at[dbc.at[j]], add=True)
        return _

      cvec = cntv[c, pl.ds(0, 16)]
      nb = jnp.minimum((jnp.max(cvec) + (K - 1)) // K, NBB)
      lax.fori_loop(0, nb, batch, None)
      plsc.subcore_barrier()
      pltpu.sync_copy(acc.at[pl.ds(tid * STRIPE, STRIPE)],
                      out.at[cid].at[pl.ds(c * CH + tid * STRIPE, STRIPE)])

  return pl.kernel(
      body,
      out_type=jax.ShapeDtypeStruct((NC, N, w), jnp.float32),
      scratch_types=[
          pltpu.VMEM((NCH, NBB, K), jnp.int32),
          pltpu.VMEM((NCH, NBB, K), jnp.int32),
          pltpu.VMEM((NCH, 16), jnp.int32),
          pltpu.VMEM((K, w), jnp.float32),
          pltpu.VMEM((K, w), jnp.float32),
          pltpu.VMEM((ZR, w), jnp.float32),
          pltpu.VMEM_SHARED((CH + 8, w), jnp.float32),
          pltpu.SemaphoreType.DMA,
          pltpu.SemaphoreType.DMA,
      ],
      **_sc_kwargs(),
  )


def _seg_tile(G):
  R = NW // G
  NWR = G
  NBT = NWR * NBB
  SBN = NWR * CAP

  def body(gt_h, sf_h, df_h, out, sb1, db1, rb0, rb1, acc, gs0, gs1):
    cid = lax.axis_index("c")
    tid = lax.axis_index("s")
    wid = cid * NS + tid
    g_ = wid // R
    r_ = wid % R
    rb = (rb0, rb1)
    gs = (gs0, gs1)

    for c in range(NCH):
      pltpu.sync_copy(sf_h.at[c].at[pl.ds(r_ * SBN, SBN)], sb1)
      pltpu.sync_copy(df_h.at[c].at[pl.ds(r_ * SBN, SBN)], db1)

      def adj(i, _):
        v = sb1[pl.ds(i * 16, 16)]
        sb1[pl.ds(i * 16, 16)] = v + jnp.broadcast_to(g_ * N, (16,))
        return _

      lax.fori_loop(0, SBN // 16, adj, None)

      def zero(i, _):
        acc[i, pl.ds(0, 16)] = jnp.zeros((16,), jnp.float32)
        return _

      lax.fori_loop(0, CH + 8, zero, None)

      pltpu.async_copy(gt_h.at[sb1.at[pl.ds(0, K)]], rb0, gs0)
      pltpu.async_copy(gt_h.at[sb1.at[pl.ds(K, K)]], rb1, gs1)

      def batch(b2, _):
        for b in range(2):
          bb = b2 * 2 + b
          pltpu.make_async_copy(gt_h.at[sb1.at[pl.ds(0, K)]], rb[b],
                                gs[b]).wait()
          for e16 in range(8):
            dvec = db1[pl.ds(bb * K + e16 * 16, 16)]
            rvec = jnp.full((16,), e16 * 16, jnp.int32) + lax.iota(
                jnp.int32, 16)
            for col in range(16):
              cvec = jnp.full((16,), col, jnp.int32)
              x = plsc.load_gather(rb[b], [rvec, cvec])
              plsc.addupdate_scatter(acc, [dvec, cvec], x)
          bn = jnp.minimum(bb + 2, NBT - 1)
          pltpu.async_copy(gt_h.at[sb1.at[pl.ds(bn * K, K)]], rb[b], gs[b])
        return _

      lax.fori_loop(0, NBT // 2, batch, None)
      for b in range(2):
        pltpu.make_async_copy(gt_h.at[sb1.at[pl.ds(0, K)]], rb[b],
                              gs[b]).wait()
      pltpu.sync_copy(acc.at[pl.ds(0, CH)],
                      out.at[r_].at[g_].at[pl.ds(c * CH, CH)])

  return pl.kernel(
      body,
      out_type=jax.ShapeDtypeStruct((R, G, N, 16), jnp.float32),
      scratch_types=[
          pltpu.VMEM((SBN,), jnp.int32),
          pltpu.VMEM((SBN,), jnp.int32),
          pltpu.VMEM((K, 16), jnp.float32),
          pltpu.VMEM((K, 16), jnp.float32),
          pltpu.VMEM((CH + 8, 16), jnp.float32),
          pltpu.SemaphoreType.DMA,
          pltpu.SemaphoreType.DMA,
      ],
      **_sc_kwargs(),
  )


def _deg_sc():
  w = 8

  def body(ones_h, db_h, cnt_h, z_h, out, db, cntv, rows0, zbuf, acc):
    cid = lax.axis_index("c")
    tid = lax.axis_index("s")
    wid = cid * NS + tid

    pltpu.sync_copy(db_h.at[wid], db)
    pltpu.sync_copy(cnt_h.at[wid], cntv)
    pltpu.sync_copy(ones_h, rows0)
    pltpu.sync_copy(z_h, zbuf)

    for c in range(NCH):
      for z in range(STRIPE // ZR):
        pltpu.sync_copy(zbuf, acc.at[pl.ds(tid * STRIPE + z * ZR, ZR)])

      dbc = db.at[c]
      plsc.subcore_barrier()

      def batch(j, _):
        pltpu.sync_copy(rows0, acc.at[dbc.at[j]], add=True)
        return _

      cvec = cntv[c, pl.ds(0, 16)]
      nb = jnp.minimum((jnp.max(cvec) + (K - 1)) // K, NBB)
      lax.fori_loop(0, nb, batch, None)
      plsc.subcore_barrier()
      pltpu.sync_copy(acc.at[pl.ds(tid * STRIPE, STRIPE)],
                      out.at[cid].at[pl.ds(c * CH + tid * STRIPE, STRIPE)])

  return pl.kernel(
      body,
      out_type=jax.ShapeDtypeStruct((NC, N, w), jnp.float32),
      scratch_types=[
          pltpu.VMEM((NCH, NBB, K), jnp.int32),
          pltpu.VMEM((NCH, 16), jnp.int32),
          pltpu.VMEM((K, w), jnp.float32),
          pltpu.VMEM((ZR, w), jnp.float32),
          pltpu.VMEM_SHARED((CH + 8, w), jnp.float32),
      ],
      **_sc_kwargs(),
  )


def _dot(a, b):
  return lax.dot_general(a, b, (((1,), (0,)), ((), ())),
                         preferred_element_type=jnp.float32)


def _ffn_a(x_ref, w_ref, a_ref, c_ref, o_ref):
  y = _dot(x_ref[...], w_ref[...]) * a_ref[...] + c_ref[...]
  o_ref[...] = jnp.maximum(y, 0.0)


def _ffn_b(h_ref, w_ref, a_ref, c_ref, o_ref):
  o_ref[...] = _dot(h_ref[...], w_ref[...]) * a_ref[...] + c_ref[...]


def _dinv_body(d_ref, o_ref):
  o_ref[...] = lax.rsqrt(d_ref[0] + d_ref[1] + 1.0)


def _fused_front(x_ref, w1_ref, a1_ref, c1_ref, w2_ref, a2_ref, c2_ref,
                 wc_ref, dv_ref, o_ref):
  h = jnp.maximum(_dot(x_ref[...], w1_ref[...]) * a1_ref[...]
                  + c1_ref[...], 0.0)
  h2 = _dot(h, w2_ref[...]) * a2_ref[...] + c2_ref[...]
  o_ref[...] = _dot(h2, wc_ref[...]) * dv_ref[:, 0:1]


def _mk_pre(G_out):
  def body(h_ref, dv_ref, wc_ref, o_ref):
    dv = dv_ref[:, 0:1]
    for g in range(G_out):
      o_ref[g] = _dot(h_ref[...], wc_ref[g]) * dv
  return body


def _mk_combine(R_in, G_in):
  def body(s_ref, g_ref, dv_ref, bc_ref, wc_ref, o_ref):
    dv = dv_ref[:, 0:1]
    cols = []
    for g in range(G_in):
      sg = s_ref[0, g]
      for r in range(1, R_in):
        sg = sg + s_ref[r, g]
      cols.append(sg + g_ref[g])
    sg_all = jnp.concatenate(cols, axis=1) if G_in > 1 else cols[0]
    a = jnp.maximum(sg_all * dv + bc_ref[...], 0.0)
    G_out = o_ref.shape[0]
    for go in range(G_out):
      o_ref[go] = _dot(a, wc_ref[go]) * dv

  return body


def _combine_body(s_ref, g_ref, dv_ref, bc_ref, wc_ref, o_ref):
  dv = dv_ref[:, 0:1]
  a = jnp.maximum((s_ref[0] + s_ref[1] + g_ref[...]) * dv + bc_ref[...], 0.0)
  o_ref[...] = _dot(a, wc_ref[...]) * dv


def _last_body(s_ref, g_ref, dv_ref, bc_ref, o_ref):
  dv = dv_ref[:, 0:1]
  o_ref[...] = jnp.maximum(
      (s_ref[0] + s_ref[1] + g_ref[...]) * dv + bc_ref[...], 0.0)


def _final_mm(a_ref, w_ref, b_ref, o_ref):
  o_ref[...] = _dot(a_ref[...], w_ref[...]) + b_ref[...]


def _pad2(m, rows, cols):
  r, c = m.shape
  if r == rows and c == cols:
    return m
  return jnp.pad(m, ((0, rows - r), (0, cols - c)))


def _full_spec(shape):
  return pl.BlockSpec(shape, lambda i: tuple(0 for _ in shape))


def _row_spec(width):
  return pl.BlockSpec((TN, width), lambda i: (i, 0))


def _s_spec(width):
  return pl.BlockSpec((NC, TN, width), lambda i: (0, i, 0))


def kernel(x, edge_index, W1, b1, g1, be1, W2, b2, g2, be2,
           Wc1, bc1, Wc2, bc2, Wc3, bc3, Wc4, bc4, Wc5, bc5, fcW, fcb):
  pad_src = jnp.zeros((EPAD,), jnp.int32)
  pad_dst = jnp.full((EPAD,), -1, jnp.int32)
  src2 = jnp.concatenate([edge_index[0], pad_src]).reshape(NW, EPW)
  dst2 = jnp.concatenate([edge_index[1], pad_dst]).reshape(NW, EPW)
  zer_i = jnp.zeros((CAP,), jnp.int32)
  tra_i = CH + (jnp.arange(CAP, dtype=jnp.int32) % 8)

  sb_h, db_h, sf_h, df_h, cnt_h = _partition_sc()(src2, dst2, zer_i, tra_i)

  def seg(w, g):
    z = jnp.zeros((ZR, w), jnp.float32)
    return _seg_sc(w)(g, sb_h, db_h, cnt_h, z)

  a1 = (g1 * _BN_S).reshape(1, 1000)
  c1 = (b1 * g1 * _BN_S + be1).reshape(1, 1000)
  a2 = _pad2((g2 * _BN_S).reshape(1, 250), 1, 256)
  c2 = _pad2((b2 * g2 * _BN_S + be2).reshape(1, 250), 1, 256)

  xp = _pad2(x, N, 256)
  W1p = _pad2(W1, 256, 1000)
  W2p = _pad2(W2, 1000, 256)
  Wc1p = _pad2(Wc1, 256, 128)

  degp = _deg_sc()(jnp.ones((K, 8), jnp.float32), db_h, cnt_h,
                   jnp.zeros((ZR, 8), jnp.float32))
  dinv = pl.pallas_call(
      _dinv_body,
      grid=(N // TN,),
      in_specs=[_s_spec(8)],
      out_specs=pl.BlockSpec((TN, 8), lambda i: (i, 0)),
      out_shape=jax.ShapeDtypeStruct((N, 8), jnp.float32),
  )(degp)

  g = pl.pallas_call(
      _fused_front,
      grid=(N // TN,),
      in_specs=[_row_spec(256), _full_spec((256, 1000)),
                _full_spec((1, 1000)), _full_spec((1, 1000)),
                _full_spec((1000, 256)), _full_spec((1, 256)),
                _full_spec((1, 256)), _full_spec((256, 128)),
                _row_spec(8)],
      out_specs=_row_spec(128),
      out_shape=jax.ShapeDtypeStruct((N, 128), jnp.float32),
  )(xp, W1p, a1, c1, W2p, a2, c2, Wc1p, dinv)

  layer_cfg = [
      (128, bc1, Wc2, 64),
      (64, bc2, Wc3, 32),
      (32, bc3, Wc4, 16),
      (16, bc4, Wc5, 8),
  ]
  for d_in, bc, wc, d_out in layer_cfg:
    sp = seg(d_in, g)
    g = pl.pallas_call(
        _combine_body,
        grid=(N // TN,),
        in_specs=[_s_spec(d_in), _row_spec(d_in), _row_spec(8),
                  _full_spec((1, d_in)), _full_spec((d_in, d_out))],
        out_specs=_row_spec(d_out),
        out_shape=jax.ShapeDtypeStruct((N, d_out), jnp.float32),
    )(sp, g, dinv, bc.reshape(1, d_in), wc)
  g5 = g

  s5 = seg(8, g5)
  u5 = pl.pallas_call(
      _last_body,
      grid=(N // TN,),
      in_specs=[_s_spec(8), _row_spec(8), _row_spec(8), _full_spec((1, 8))],
      out_specs=_row_spec(8),
      out_shape=jax.ShapeDtypeStruct((N, 8), jnp.float32),
  )(s5, g5, dinv, bc5.reshape(1, 8))

  ar = u5.reshape(320, 1200)
  fcWp = _pad2(fcW, 1200, 128)
  fcbp = _pad2(fcb.reshape(1, 4), 1, 128)
  out = pl.pallas_call(
      _final_mm,
      grid=(1,),
      in_specs=[_full_spec((320, 1200)), _full_spec((1200, 128)),
                _full_spec((1, 128))],
      out_specs=pl.BlockSpec((320, 128), lambda i: (0, 0)),
      out_shape=jax.ShapeDtypeStruct((320, 128), jnp.float32),
  )(ar, fcWp, fcbp)
  return out[:, :4]
